# baseline (speedup 1.0000x reference)
import functools

import jax
import jax.numpy as jnp
from jax import lax
from jax.experimental import pallas as pl
from jax.experimental.pallas import tpu as pltpu
from jax.experimental.pallas import tpu_sc as plsc

N_FIELDS = 26
FIELD_SIZE = 100000
D = 32
B = 16384
L = 16
NC, NS = 2, 16
NW = NC * NS
BPW = B // NW
CB = 8
CB26 = CB * N_FIELDS
NCH = BPW // CB
SUB = 104
NSUB = CB26 // SUB
INV_N = float(1.0 / N_FIELDS)


def _body(idx_hbm, table_hbm, out_hbm, idx_v, rows_v, out_v, sem):
    wid = lax.axis_index("s") * NC + lax.axis_index("c")
    ibase = wid * (BPW * N_FIELDS)

    def chunk_body(c, carry):
        cbase = ibase + c * CB26
        pltpu.sync_copy(idx_hbm.at[pl.ds(cbase, CB26)], idx_v)
        for v in range(CB26 // L):
            k0 = v * L
            off = ((lax.iota(jnp.int32, L) + k0) % N_FIELDS) * FIELD_SIZE
            idx_v[pl.ds(k0, L)] = idx_v[pl.ds(k0, L)] + off
        handles = []
        for s in range(NSUB):
            handles.append(pltpu.async_copy(
                table_hbm.at[idx_v.at[pl.ds(s * SUB, SUB)]],
                rows_v.at[pl.ds(s * SUB, SUB)], sem))
        for h in handles:
            h.wait()
        for i in range(CB):
            r0 = i * N_FIELDS
            acc0 = rows_v[r0, pl.ds(0, L)]
            acc1 = rows_v[r0, pl.ds(L, L)]
            for j in range(1, N_FIELDS):
                acc0 = acc0 + rows_v[r0 + j, pl.ds(0, L)]
                acc1 = acc1 + rows_v[r0 + j, pl.ds(L, L)]
            orow = c * CB + i
            out_v[orow, pl.ds(0, L)] = acc0 * INV_N
            out_v[orow, pl.ds(L, L)] = acc1 * INV_N
        return carry

    lax.fori_loop(0, NCH, chunk_body, 0)
    pltpu.sync_copy(out_v, out_hbm.at[pl.ds(wid * BPW, BPW)])


@jax.jit
def _sc_embed(idx_flat, table):
    mesh = plsc.VectorSubcoreMesh(core_axis_name="c", subcore_axis_name="s")
    return pl.kernel(
        _body,
        out_type=jax.ShapeDtypeStruct((B, D), jnp.float32),
        mesh=mesh,
        scratch_types=[
            pltpu.VMEM((CB26,), jnp.int32),
            pltpu.VMEM((CB26, D), jnp.float32),
            pltpu.VMEM((BPW, D), jnp.float32),
            pltpu.SemaphoreType.DMA,
        ],
        compiler_params=pltpu.CompilerParams(use_tc_tiling_on_sc=False),
    )(idx_flat, table)


def kernel(inputs, embedding):
    return _sc_embed(inputs.reshape(-1), embedding)


# speedup vs baseline: 1.0329x; 1.0329x over previous
import jax
import jax.numpy as jnp
from jax import lax
from jax.experimental import pallas as pl
from jax.experimental.pallas import tpu as pltpu
from jax.experimental.pallas import tpu_sc as plsc

N_FIELDS = 26
FIELD_SIZE = 100000
D = 32
B = 16384
L = 16
NC, NS = 2, 16
NW = NC * NS
BPW = B // NW
IPW = BPW * N_FIELDS
GROWS = 104
GB = GROWS // N_FIELDS
NG = IPW // GROWS
R = 8
NITER = NG // R
PERIOD = 208
INV_N = float(1.0 / N_FIELDS)


def _fire(table_hbm, idx_v, rows_v, sem, g, b):
    src = table_hbm.at[idx_v.at[pl.ds(g * GROWS, GROWS)]]
    return pltpu.async_copy(src, rows_v.at[b], sem.at[b])


def _body(idx_hbm, table_hbm, out_hbm, idx_v, rows_v, out_v, sem):
    wid = lax.axis_index("s") * NC + lax.axis_index("c")

    pltpu.sync_copy(idx_hbm.at[pl.ds(wid * IPW, IPW)], idx_v)

    def off_body(blk, carry):
        base = blk * PERIOD
        for v in range(PERIOD // L):
            off = ((lax.iota(jnp.int32, L) + v * L) % N_FIELDS) * FIELD_SIZE
            sl = pl.ds(base + v * L, L)
            idx_v[sl] = idx_v[sl] + off
        return carry

    lax.fori_loop(0, IPW // PERIOD, off_body, 0)

    for b in range(R):
        _fire(table_hbm, idx_v, rows_v, sem, b, b)

    def ring_body(i, carry):
        for b in range(R):
            g = i * R + b
            pltpu.make_async_copy(
                table_hbm.at[idx_v.at[pl.ds(0, GROWS)]],
                rows_v.at[b], sem.at[b]).wait()
            for ii in range(GB):
                r0 = ii * N_FIELDS
                acc0 = rows_v[b, r0, pl.ds(0, L)]
                acc1 = rows_v[b, r0, pl.ds(L, L)]
                for j in range(1, N_FIELDS):
                    acc0 = acc0 + rows_v[b, r0 + j, pl.ds(0, L)]
                    acc1 = acc1 + rows_v[b, r0 + j, pl.ds(L, L)]
                orow = g * GB + ii
                out_v[orow, pl.ds(0, L)] = acc0 * INV_N
                out_v[orow, pl.ds(L, L)] = acc1 * INV_N

            @pl.when(i + 1 < NITER)
            def _():
                _fire(table_hbm, idx_v, rows_v, sem, g + R, b)
        return carry

    lax.fori_loop(0, NITER, ring_body, 0)

    pltpu.sync_copy(out_v, out_hbm.at[pl.ds(wid * BPW, BPW)])


@jax.jit
def _sc_embed(idx_flat, table):
    mesh = plsc.VectorSubcoreMesh(core_axis_name="c", subcore_axis_name="s")
    return pl.kernel(
        _body,
        out_type=jax.ShapeDtypeStruct((B, D), jnp.float32),
        mesh=mesh,
        scratch_types=[
            pltpu.VMEM((IPW,), jnp.int32),
            pltpu.VMEM((R, GROWS, D), jnp.float32),
            pltpu.VMEM((BPW, D), jnp.float32),
            pltpu.SemaphoreType.DMA((R,)),
        ],
        compiler_params=pltpu.CompilerParams(use_tc_tiling_on_sc=False),
    )(idx_flat, table)


def kernel(inputs, embedding):
    return _sc_embed(inputs.reshape(-1), embedding)
